# baseline (speedup 1.0000x reference)
import functools

import jax
import jax.numpy as jnp
from jax import lax
from jax.experimental import pallas as pl
from jax.experimental.pallas import tpu as pltpu
from jax.experimental.pallas import tpu_sc as plsc

B = 16384
D = 32
N_CORES = 2
N_SUBCORES = 16
NW = N_CORES * N_SUBCORES
BPW = B // NW
IDX_CHUNK = 128
NCHUNK = BPW // IDX_CHUNK
LANES = 16

_mesh = plsc.VectorSubcoreMesh(core_axis_name="c", subcore_axis_name="s")


@functools.partial(
    pl.kernel,
    mesh=_mesh,
    out_type=jax.ShapeDtypeStruct((B,), jnp.float32),
    compiler_params=pltpu.CompilerParams(
        needs_layout_passes=False, use_tc_tiling_on_sc=False),
    scratch_types=[
        pltpu.VMEM((NCHUNK, IDX_CHUNK), jnp.int32),
        pltpu.VMEM((NCHUNK, IDX_CHUNK), jnp.int32),
        pltpu.VMEM((BPW, D), jnp.float32),
        pltpu.VMEM((BPW, D), jnp.float32),
        pltpu.VMEM((BPW,), jnp.float32),
        pltpu.SemaphoreType.DMA,
    ],
)
def _gcmc_sc_kernel(uid_hbm, iid_hbm, utab_hbm, itab_hbm, out_hbm,
                    uidx, iidx, urows, irows, out_v, sem):
    wid = lax.axis_index("s") * N_CORES + lax.axis_index("c")
    base = wid * BPW

    pltpu.sync_copy(uid_hbm.at[wid], uidx)
    pltpu.sync_copy(iid_hbm.at[wid], iidx)

    copies = []
    for j in range(NCHUNK):
        copies.append(pltpu.async_copy(
            utab_hbm.at[uidx.at[j]],
            urows.at[pl.ds(j * IDX_CHUNK, IDX_CHUNK)], sem))
        copies.append(pltpu.async_copy(
            itab_hbm.at[iidx.at[j]],
            irows.at[pl.ds(j * IDX_CHUNK, IDX_CHUNK)], sem))
    for c in copies:
        c.wait()

    lane = jnp.arange(LANES, dtype=jnp.int32)

    def block_body(b, carry):
        row0 = b * LANES
        acc = jnp.zeros((LANES,), jnp.float32)
        for i in range(LANES):
            r = row0 + i
            p = (urows[r, pl.ds(0, LANES)] * irows[r, pl.ds(0, LANES)]
                 + urows[r, pl.ds(LANES, LANES)] * irows[r, pl.ds(LANES, LANES)])
            acc = jnp.where(lane == i, jnp.sum(p), acc)
        out_v[pl.ds(row0, LANES)] = acc
        return carry

    lax.fori_loop(0, BPW // LANES, block_body, 0)

    pltpu.sync_copy(out_v, out_hbm.at[pl.ds(base, BPW)])


def kernel(x, user_embedding, item_embedding):
    uid = x[:, 0].reshape(NW, NCHUNK, IDX_CHUNK)
    iid = x[:, 1].reshape(NW, NCHUNK, IDX_CHUNK)
    return _gcmc_sc_kernel(uid, iid, user_embedding, item_embedding)


# speedup vs baseline: 4.4215x; 4.4215x over previous
import functools

import jax
import jax.numpy as jnp
from jax import lax
from jax.experimental import pallas as pl
from jax.experimental.pallas import tpu as pltpu
from jax.experimental.pallas import tpu_sc as plsc

B = 16384
D = 32
N_CORES = 2
N_SUBCORES = 16
NW = N_CORES * N_SUBCORES
BPW = B // NW
LANES = 16
NBUF = 8

_mesh = plsc.VectorSubcoreMesh(core_axis_name="c", subcore_axis_name="s")


@functools.partial(
    pl.kernel,
    mesh=_mesh,
    out_type=jax.ShapeDtypeStruct((B,), jnp.float32),
    compiler_params=pltpu.CompilerParams(
        needs_layout_passes=False, use_tc_tiling_on_sc=True),
    scratch_types=[
        pltpu.VMEM((BPW + LANES,), jnp.int32),
        pltpu.VMEM((BPW + LANES,), jnp.int32),
        pltpu.VMEM((NBUF, D, 128), jnp.float32),
        pltpu.VMEM((NBUF, D, 128), jnp.float32),
        pltpu.VMEM((BPW,), jnp.float32),
        pltpu.SemaphoreType.DMA,
        pltpu.SemaphoreType.DMA,
    ],
)
def _gcmc_sc_kernel(uid_hbm, iid_hbm, utT_hbm, itT_hbm, out_hbm,
                    uidx, iidx, ublk, iblk, out_v, usem, isem):
    wid = lax.axis_index("s") * N_CORES + lax.axis_index("c")
    base = wid * BPW

    pltpu.sync_copy(uid_hbm.at[pl.ds(base, BPW)], uidx.at[pl.ds(0, BPW)])
    pltpu.sync_copy(iid_hbm.at[pl.ds(base, BPW)], iidx.at[pl.ds(0, BPW)])

    lane = jnp.arange(LANES, dtype=jnp.int32)
    c_lo = lane * 128
    c_hi = c_lo + LANES * 128

    def issue(uvi, ivi, slot):
        tc_u = pl.multiple_of((uvi >> 7) * 128, 128)
        tc_i = pl.multiple_of((ivi >> 7) * 128, 128)
        pltpu.make_async_copy(
            utT_hbm.at[:, pl.ds(tc_u, 128)], ublk.at[slot], usem).start()
        pltpu.make_async_copy(
            itT_hbm.at[:, pl.ds(tc_i, 128)], iblk.at[slot], isem).start()

    def wait(slot):
        pltpu.make_async_copy(
            utT_hbm.at[:, pl.ds(0, 128)], ublk.at[slot], usem).wait()
        pltpu.make_async_copy(
            itT_hbm.at[:, pl.ds(0, 128)], iblk.at[slot], isem).wait()

    uvec0 = uidx[pl.ds(0, LANES)]
    ivec0 = iidx[pl.ds(0, LANES)]
    for n in range(NBUF):
        issue(uvec0[n], ivec0[n], n)

    def body(g, carry):
        uvec = uidx[pl.ds(g * LANES, LANES)]
        ivec = iidx[pl.ds(g * LANES, LANES)]
        uvec_n = uidx[pl.ds((g + 1) * LANES, LANES)]
        ivec_n = iidx[pl.ds((g + 1) * LANES, LANES)]
        acc = jnp.zeros((LANES,), jnp.float32)
        for i in range(LANES):
            slot = i % NBUF
            lu = jnp.full((LANES,), uvec[i] & 127, jnp.int32)
            li = jnp.full((LANES,), ivec[i] & 127, jnp.int32)

            wait(slot)
            u0 = plsc.load_gather(ublk.at[slot], [lane, lu])
            u1 = plsc.load_gather(ublk.at[slot], [lane + LANES, lu])
            v0 = plsc.load_gather(iblk.at[slot], [lane, li])
            v1 = plsc.load_gather(iblk.at[slot], [lane + LANES, li])
            s = jnp.sum(u0 * v0 + u1 * v1)
            acc = jnp.where(lane == i, s, acc)

            if i + NBUF < LANES:
                issue(uvec[i + NBUF], ivec[i + NBUF], slot)
            else:
                @pl.when(g < BPW // LANES - 1)
                def _():
                    issue(uvec_n[i + NBUF - LANES],
                          ivec_n[i + NBUF - LANES], slot)

        out_v[pl.ds(g * LANES, LANES)] = acc
        return carry

    lax.fori_loop(0, BPW // LANES, body, 0)

    pltpu.sync_copy(out_v, out_hbm.at[pl.ds(base, BPW)])


def kernel(x, user_embedding, item_embedding):
    uid = x[:, 0]
    iid = x[:, 1]
    return _gcmc_sc_kernel(uid, iid, user_embedding.T, item_embedding.T)
